# Initial kernel scaffold; baseline (speedup 1.0000x reference)
import jax
import jax.numpy as jnp
from jax.experimental import pallas as pl


def kernel(x, edge_attr, W, b, edge_index):
    raise NotImplementedError("write your pallas kernel here")


# speedup vs baseline: 7.2293x; 7.2293x over previous
import functools

import jax
import jax.numpy as jnp
from jax import lax
from jax.experimental import pallas as pl
from jax.experimental.pallas import tpu as pltpu
from jax.experimental.pallas import tpu_sc as plsc

N_NODES = 10000
N_EDGES = 320000
D_FEAT = 128
D_EDGE = 16

NC = 2
NS = 16
NW = NC * NS

E_PER_W = N_EDGES // NW
BLK = 80
INNER = 5
SUPER = BLK * INNER
N_OUTER = E_PER_W // SUPER
ROWS_PER_W = N_NODES // NS


def _sc_body(edge_hbm, idx_hbm, in_part_hbm, out_part_hbm,
             idx_v, blk_v, z_v, in_sh, out_sh):
    c = lax.axis_index("c")
    s = lax.axis_index("s")
    wid = c * NS + s
    base_e = wid * E_PER_W

    pltpu.sync_copy(idx_hbm.at[wid], idx_v)

    def _zero(i, carry):
        z_v[i] = jnp.zeros((D_EDGE,), jnp.float32)
        return carry
    lax.fori_loop(0, ROWS_PER_W, _zero, 0)
    row0 = s * ROWS_PER_W
    pltpu.sync_copy(z_v, in_sh.at[pl.ds(row0, ROWS_PER_W)])
    pltpu.sync_copy(z_v, out_sh.at[pl.ds(row0, ROWS_PER_W)])
    plsc.subcore_barrier()

    def _outer(j, carry):
        pltpu.sync_copy(edge_hbm.at[pl.ds(base_e + j * SUPER, SUPER)], blk_v)
        for u in range(INNER):
            blk = blk_v.at[pl.ds(u * BLK, BLK)]
            jj = j * INNER + u
            pltpu.sync_copy(blk, in_sh.at[idx_v.at[1, jj]], add=True)
            pltpu.sync_copy(blk, out_sh.at[idx_v.at[0, jj]], add=True)
        return carry
    lax.fori_loop(0, N_OUTER, _outer, 0)
    plsc.subcore_barrier()

    pltpu.sync_copy(in_sh.at[pl.ds(row0, ROWS_PER_W)], in_part_hbm.at[c, s])
    pltpu.sync_copy(out_sh.at[pl.ds(row0, ROWS_PER_W)], out_part_hbm.at[c, s])


_sc_agg = pl.kernel(
    _sc_body,
    out_type=(
        jax.ShapeDtypeStruct((NC, NS, ROWS_PER_W, D_EDGE), jnp.float32),
        jax.ShapeDtypeStruct((NC, NS, ROWS_PER_W, D_EDGE), jnp.float32),
    ),
    mesh=plsc.VectorSubcoreMesh(core_axis_name="c", subcore_axis_name="s"),
    compiler_params=pltpu.CompilerParams(use_tc_tiling_on_sc=False),
    scratch_types=[
        pltpu.VMEM((2, N_OUTER * INNER, BLK), jnp.int32),
        pltpu.VMEM((SUPER, D_EDGE), jnp.float32),
        pltpu.VMEM((ROWS_PER_W, D_EDGE), jnp.float32),
        pltpu.VMEM_SHARED((N_NODES, D_EDGE), jnp.float32),
        pltpu.VMEM_SHARED((N_NODES, D_EDGE), jnp.float32),
    ],
)


ROW_BLK = 1000


def _mlp_body(inp_ref, outp_ref, x_ref, wi_ref, wo_ref, wx_ref, b_ref, o_ref):
    a = inp_ref[0] + inp_ref[1]
    o = outp_ref[0] + outp_ref[1]
    acc = jnp.dot(a, wi_ref[...], preferred_element_type=jnp.float32)
    acc = acc + jnp.dot(o, wo_ref[...], preferred_element_type=jnp.float32)
    acc = acc + jnp.dot(x_ref[...], wx_ref[...], preferred_element_type=jnp.float32)
    o_ref[...] = acc + b_ref[...]


@functools.partial(jax.jit, static_argnames=())
def kernel(x, edge_attr, W, b, edge_index):
    idx32 = edge_index.astype(jnp.int32)
    idx_t = idx32.reshape(2, NW, N_OUTER * INNER, BLK).transpose(1, 0, 2, 3)
    in_part, out_part = _sc_agg(edge_attr, idx_t)
    in_part = in_part.reshape(NC, N_NODES, D_EDGE)
    out_part = out_part.reshape(NC, N_NODES, D_EDGE)

    wi = W[:D_EDGE]
    wo = W[D_EDGE:2 * D_EDGE]
    wx = W[2 * D_EDGE:]
    b2 = b.reshape(1, D_FEAT)

    grid = (N_NODES // ROW_BLK,)
    out = pl.pallas_call(
        _mlp_body,
        grid=grid,
        in_specs=[
            pl.BlockSpec((NC, ROW_BLK, D_EDGE), lambda i: (0, i, 0)),
            pl.BlockSpec((NC, ROW_BLK, D_EDGE), lambda i: (0, i, 0)),
            pl.BlockSpec((ROW_BLK, D_FEAT), lambda i: (i, 0)),
            pl.BlockSpec((D_EDGE, D_FEAT), lambda i: (0, 0)),
            pl.BlockSpec((D_EDGE, D_FEAT), lambda i: (0, 0)),
            pl.BlockSpec((D_FEAT, D_FEAT), lambda i: (0, 0)),
            pl.BlockSpec((1, D_FEAT), lambda i: (0, 0)),
        ],
        out_specs=pl.BlockSpec((ROW_BLK, D_FEAT), lambda i: (i, 0)),
        out_shape=jax.ShapeDtypeStruct((N_NODES, D_FEAT), jnp.float32),
    )(in_part, out_part, x, wi, wo, wx, b2)
    return out


# speedup vs baseline: 8.4851x; 1.1737x over previous
import functools

import jax
import jax.numpy as jnp
from jax import lax
from jax.experimental import pallas as pl
from jax.experimental.pallas import tpu as pltpu
from jax.experimental.pallas import tpu_sc as plsc

N_NODES = 10000
N_EDGES = 320000
D_FEAT = 128
D_EDGE = 16

NC = 2
NS = 16
NW = NC * NS

E_PER_W = N_EDGES // NW
BLK = 80
INNER = 5
SUPER = BLK * INNER
N_OUTER = E_PER_W // SUPER
ROWS_PER_W = N_NODES // NS


NBUF = 4


def _sc_body(edge_hbm, idx_hbm, in_part_hbm, out_part_hbm,
             idx_v, blk_v, z_v, in_sh, out_sh, f_sems, a_sems):
    c = lax.axis_index("c")
    s = lax.axis_index("s")
    wid = c * NS + s
    base_e = wid * E_PER_W

    pltpu.sync_copy(idx_hbm.at[0, wid], idx_v.at[0])
    pltpu.sync_copy(idx_hbm.at[1, wid], idx_v.at[1])

    def _zero(i, carry):
        z_v[i] = jnp.zeros((D_EDGE,), jnp.float32)
        return carry
    lax.fori_loop(0, ROWS_PER_W, _zero, 0)
    row0 = s * ROWS_PER_W
    pltpu.sync_copy(z_v, in_sh.at[pl.ds(row0, ROWS_PER_W)])
    pltpu.sync_copy(z_v, out_sh.at[pl.ds(row0, ROWS_PER_W)])
    plsc.subcore_barrier()

    def _fetch(j, p):
        pltpu.async_copy(edge_hbm.at[pl.ds(base_e + j * SUPER, SUPER)],
                         blk_v.at[p], f_sems.at[p])

    def _drain_scatters(p):
        pltpu.make_async_copy(edge_hbm.at[pl.ds(0, SUPER)], blk_v.at[p],
                              a_sems.at[p]).wait()
        pltpu.make_async_copy(edge_hbm.at[pl.ds(0, SUPER)], blk_v.at[p],
                              a_sems.at[p]).wait()

    _fetch(0, 0)
    _fetch(1, 1)

    def _outer(t, carry):
        p = lax.rem(t, NBUF)
        pn = lax.rem(t + 2, NBUF)

        @pl.when(t >= 2)
        def _():
            _drain_scatters(pn)

        @pl.when(t + 2 < N_OUTER)
        def _():
            _fetch(t + 2, pn)

        pltpu.make_async_copy(edge_hbm.at[pl.ds(base_e + t * SUPER, SUPER)],
                              blk_v.at[p], f_sems.at[p]).wait()
        for u in range(INNER):
            blk = blk_v.at[p].at[pl.ds(u * BLK, BLK)]
            jj = t * INNER + u
            pltpu.async_copy(blk, in_sh.at[idx_v.at[1, jj]], a_sems.at[p],
                             add=True)
            pltpu.async_copy(blk, out_sh.at[idx_v.at[0, jj]], a_sems.at[p],
                             add=True)
        return carry
    lax.fori_loop(0, N_OUTER, _outer, 0)
    _drain_scatters(lax.rem(jnp.int32(N_OUTER - 2), NBUF))
    _drain_scatters(lax.rem(jnp.int32(N_OUTER - 1), NBUF))
    plsc.subcore_barrier()

    pltpu.sync_copy(in_sh.at[pl.ds(row0, ROWS_PER_W)], in_part_hbm.at[c, s])
    pltpu.sync_copy(out_sh.at[pl.ds(row0, ROWS_PER_W)], out_part_hbm.at[c, s])


_sc_agg = pl.kernel(
    _sc_body,
    out_type=(
        jax.ShapeDtypeStruct((NC, NS, ROWS_PER_W, D_EDGE), jnp.float32),
        jax.ShapeDtypeStruct((NC, NS, ROWS_PER_W, D_EDGE), jnp.float32),
    ),
    mesh=plsc.VectorSubcoreMesh(core_axis_name="c", subcore_axis_name="s"),
    compiler_params=pltpu.CompilerParams(use_tc_tiling_on_sc=False),
    scratch_types=[
        pltpu.VMEM((2, N_OUTER * INNER, BLK), jnp.int32),
        pltpu.VMEM((NBUF, SUPER, D_EDGE), jnp.float32),
        pltpu.VMEM((ROWS_PER_W, D_EDGE), jnp.float32),
        pltpu.VMEM_SHARED((N_NODES, D_EDGE), jnp.float32),
        pltpu.VMEM_SHARED((N_NODES, D_EDGE), jnp.float32),
        pltpu.SemaphoreType.DMA((NBUF,)),
        pltpu.SemaphoreType.DMA((NBUF,)),
    ],
)


ROW_BLK = 1000


def _mlp_body(inp_ref, outp_ref, x_ref, wi_ref, wo_ref, wx_ref, b_ref, o_ref):
    a = inp_ref[0] + inp_ref[1]
    o = outp_ref[0] + outp_ref[1]
    acc = jnp.dot(a, wi_ref[...], preferred_element_type=jnp.float32)
    acc = acc + jnp.dot(o, wo_ref[...], preferred_element_type=jnp.float32)
    acc = acc + jnp.dot(x_ref[...], wx_ref[...], preferred_element_type=jnp.float32)
    o_ref[...] = acc + b_ref[...]


@functools.partial(jax.jit, static_argnames=())
def kernel(x, edge_attr, W, b, edge_index):
    idx32 = edge_index.astype(jnp.int32)
    idx_t = idx32.reshape(2, NW, N_OUTER * INNER, BLK)
    in_part, out_part = _sc_agg(edge_attr, idx_t)
    in_part = in_part.reshape(NC, N_NODES, D_EDGE)
    out_part = out_part.reshape(NC, N_NODES, D_EDGE)

    wi = W[:D_EDGE]
    wo = W[D_EDGE:2 * D_EDGE]
    wx = W[2 * D_EDGE:]
    b2 = b.reshape(1, D_FEAT)

    grid = (N_NODES // ROW_BLK,)
    out = pl.pallas_call(
        _mlp_body,
        grid=grid,
        in_specs=[
            pl.BlockSpec((NC, ROW_BLK, D_EDGE), lambda i: (0, i, 0)),
            pl.BlockSpec((NC, ROW_BLK, D_EDGE), lambda i: (0, i, 0)),
            pl.BlockSpec((ROW_BLK, D_FEAT), lambda i: (i, 0)),
            pl.BlockSpec((D_EDGE, D_FEAT), lambda i: (0, 0)),
            pl.BlockSpec((D_EDGE, D_FEAT), lambda i: (0, 0)),
            pl.BlockSpec((D_FEAT, D_FEAT), lambda i: (0, 0)),
            pl.BlockSpec((1, D_FEAT), lambda i: (0, 0)),
        ],
        out_specs=pl.BlockSpec((ROW_BLK, D_FEAT), lambda i: (i, 0)),
        out_shape=jax.ShapeDtypeStruct((N_NODES, D_FEAT), jnp.float32),
    )(in_part, out_part, x, wi, wo, wx, b2)
    return out


# speedup vs baseline: 10.3650x; 1.2215x over previous
import functools

import jax
import jax.numpy as jnp
from jax import lax
from jax.experimental import pallas as pl
from jax.experimental.pallas import tpu as pltpu
from jax.experimental.pallas import tpu_sc as plsc

N_NODES = 10000
N_EDGES = 320000
D_FEAT = 128
D_EDGE = 16

NC = 2
NS = 16

N_EB = N_EDGES // 128
CH = 125
N_CH = N_EB // CH
N_PAD = 10240
NRB = N_PAD // 128


def _sc_body(e4_hbm, idx_hbm, out_hbm,
             idx_sh, val_v, idx_v, acc, stg, vs0, vs1, is0, is1):
    c = lax.axis_index("c")
    s = lax.axis_index("s")
    fh = s // 8
    fm = lax.rem(s, 8)

    @pl.when(s == 0)
    def _():
        pltpu.sync_copy(idx_hbm.at[1 - c], idx_sh)

    def _zero(i, carry):
        acc[pl.ds(i * 16, 16)] = jnp.zeros((16,), jnp.float32)
        return carry
    lax.fori_loop(0, N_PAD // 16, _zero, 0)
    plsc.subcore_barrier()

    vsems = (vs0, vs1)
    isems = (is0, is1)

    def _fetch(ch, b):
        pltpu.async_copy(
            e4_hbm.at[fh, pl.ds(ch * CH, CH), pl.ds(fm, 1), :],
            val_v.at[b], vsems[b])
        pltpu.async_copy(idx_sh.at[pl.ds(ch * CH, CH)], idx_v.at[b], isems[b])

    def _process(b):
        pltpu.make_async_copy(
            e4_hbm.at[fh, pl.ds(0, CH), pl.ds(fm, 1), :],
            val_v.at[b], vsems[b]).wait()
        pltpu.make_async_copy(idx_sh.at[pl.ds(0, CH)], idx_v.at[b],
                              isems[b]).wait()

        def _row(r, carry):
            for u in range(8):
                iv = idx_v[b, r, pl.ds(u * 16, 16)]
                vv = val_v[b, r, 0, pl.ds(u * 16, 16)]
                plsc.addupdate_scatter(acc, [iv], vv)
            return carry
        lax.fori_loop(0, CH, _row, 0)

    _fetch(0, 0)
    _fetch(1, 1)

    def _pair(t, carry):
        _process(0)

        @pl.when(2 * t + 2 < N_CH)
        def _():
            _fetch(2 * t + 2, 0)
        _process(1)

        @pl.when(2 * t + 3 < N_CH)
        def _():
            _fetch(2 * t + 3, 1)
        return carry
    lax.fori_loop(0, N_CH // 2, _pair, 0)

    def _rp(r, carry):
        for u in range(8):
            stg[r, pl.ds(u * 16, 16)] = acc[pl.ds(r * 128 + u * 16, 16)]
        return carry
    lax.fori_loop(0, NRB, _rp, 0)
    pltpu.sync_copy(stg, out_hbm.at[c, s])


_sc_agg = pl.kernel(
    _sc_body,
    out_type=jax.ShapeDtypeStruct((NC, NS, NRB, 128), jnp.float32),
    mesh=plsc.VectorSubcoreMesh(core_axis_name="c", subcore_axis_name="s"),
    compiler_params=pltpu.CompilerParams(use_tc_tiling_on_sc=False,
                                         needs_layout_passes=False),
    scratch_types=[
        pltpu.VMEM_SHARED((N_EB, 128), jnp.int32),
        pltpu.VMEM((2, CH, 1, 128), jnp.float32),
        pltpu.VMEM((2, CH, 128), jnp.int32),
        pltpu.VMEM((N_PAD,), jnp.float32),
        pltpu.VMEM((NRB, 128), jnp.float32),
        pltpu.SemaphoreType.DMA,
        pltpu.SemaphoreType.DMA,
        pltpu.SemaphoreType.DMA,
        pltpu.SemaphoreType.DMA,
    ],
)


ROW_BLK = 1024


def _mlp_body(agg_ref, x_ref, wio_ref, wx_ref, b_ref, o_ref):
    base = jnp.dot(x_ref[...], wx_ref[...],
                   preferred_element_type=jnp.float32) + b_ref[...]
    ac = agg_ref[...].reshape(2 * D_EDGE, 8, 128)
    for j in range(8):
        aj = ac[:, j, :]
        part = lax.dot_general(aj, wio_ref[...], (((0,), (0,)), ((), ())),
                               preferred_element_type=jnp.float32)
        o_ref[pl.ds(j * 128, 128), :] = base[j * 128:(j + 1) * 128, :] + part


@functools.partial(jax.jit, static_argnames=())
def kernel(x, edge_attr, W, b, edge_index):
    e4 = edge_attr.T.reshape(NC, 8, N_EB, 128).transpose(0, 2, 1, 3)
    idx3 = edge_index.astype(jnp.int32).reshape(2, N_EB, 128)
    agg = _sc_agg(e4, idx3)

    wio = W[:2 * D_EDGE]
    wx = W[2 * D_EDGE:]
    b2 = b.reshape(1, D_FEAT)

    grid = (N_PAD // ROW_BLK,)
    out = pl.pallas_call(
        _mlp_body,
        grid=grid,
        in_specs=[
            pl.BlockSpec((NC, NS, 8, 128), lambda i: (0, 0, i, 0)),
            pl.BlockSpec((ROW_BLK, D_FEAT), lambda i: (i, 0)),
            pl.BlockSpec((2 * D_EDGE, D_FEAT), lambda i: (0, 0)),
            pl.BlockSpec((D_FEAT, D_FEAT), lambda i: (0, 0)),
            pl.BlockSpec((1, D_FEAT), lambda i: (0, 0)),
        ],
        out_specs=pl.BlockSpec((ROW_BLK, D_FEAT), lambda i: (i, 0)),
        out_shape=jax.ShapeDtypeStruct((N_NODES, D_FEAT), jnp.float32),
    )(agg, x, wio, wx, b2)
    return out


# speedup vs baseline: 18.4722x; 1.7822x over previous
import functools

import jax
import jax.numpy as jnp
from jax import lax
from jax.experimental import pallas as pl
from jax.experimental.pallas import tpu as pltpu
from jax.experimental.pallas import tpu_sc as plsc

N_NODES = 10000
N_EDGES = 320000
D_FEAT = 128
D_EDGE = 16

NC = 2
NS = 16

N_EB = N_EDGES // 128
CH = 125
N_CH = N_EB // CH
N_PAD = 10240
NRB = N_PAD // 128


def _sc_body(e4_hbm, idx_hbm, out_hbm,
             idx_sh, val_v, idx_v, acc, stg, vs0, vs1, is0, is1):
    c = lax.axis_index("c")
    s = lax.axis_index("s")
    fh = s // 8
    fm = lax.rem(s, 8)

    @pl.when(s == 0)
    def _():
        pltpu.sync_copy(idx_hbm.at[1 - c], idx_sh)

    def _zero(i, carry):
        acc[pl.ds(i * 16, 16)] = jnp.zeros((16,), jnp.float32)
        return carry
    lax.fori_loop(0, N_PAD // 16, _zero, 0)
    plsc.subcore_barrier()

    vsems = (vs0, vs1)
    isems = (is0, is1)

    def _fetch(ch, b):
        pltpu.async_copy(
            e4_hbm.at[fh, pl.ds(ch * CH, CH), pl.ds(fm, 1), :],
            val_v.at[b], vsems[b])
        pltpu.async_copy(idx_sh.at[pl.ds(ch * CH, CH)], idx_v.at[b], isems[b])

    def _process(b):
        pltpu.make_async_copy(
            e4_hbm.at[fh, pl.ds(0, CH), pl.ds(fm, 1), :],
            val_v.at[b], vsems[b]).wait()
        pltpu.make_async_copy(idx_sh.at[pl.ds(0, CH)], idx_v.at[b],
                              isems[b]).wait()

        def _row(r, carry):
            ivs = [idx_v[b, r, pl.ds(u * 16, 16)] for u in range(8)]
            vvs = [val_v[b, r, 0, pl.ds(u * 16, 16)] for u in range(8)]
            for u in range(8):
                plsc.addupdate_scatter(acc, [ivs[u]], vvs[u])
            return carry
        lax.fori_loop(0, CH, _row, 0)

    _fetch(0, 0)
    _fetch(1, 1)

    def _pair(t, carry):
        _process(0)

        @pl.when(2 * t + 2 < N_CH)
        def _():
            _fetch(2 * t + 2, 0)
        _process(1)

        @pl.when(2 * t + 3 < N_CH)
        def _():
            _fetch(2 * t + 3, 1)
        return carry
    lax.fori_loop(0, N_CH // 2, _pair, 0)

    def _rp(r, carry):
        for u in range(8):
            stg[r, pl.ds(u * 16, 16)] = acc[pl.ds(r * 128 + u * 16, 16)]
        return carry
    lax.fori_loop(0, NRB, _rp, 0)
    pltpu.sync_copy(stg, out_hbm.at[c, s])


_sc_agg = pl.kernel(
    _sc_body,
    out_type=jax.ShapeDtypeStruct((NC, NS, NRB, 128), jnp.float32),
    mesh=plsc.VectorSubcoreMesh(core_axis_name="c", subcore_axis_name="s"),
    compiler_params=pltpu.CompilerParams(use_tc_tiling_on_sc=False,
                                         needs_layout_passes=False),
    scratch_types=[
        pltpu.VMEM_SHARED((N_EB, 128), jnp.int32),
        pltpu.VMEM((2, CH, 1, 128), jnp.float32),
        pltpu.VMEM((2, CH, 128), jnp.int32),
        pltpu.VMEM((N_PAD,), jnp.float32),
        pltpu.VMEM((NRB, 128), jnp.float32),
        pltpu.SemaphoreType.DMA,
        pltpu.SemaphoreType.DMA,
        pltpu.SemaphoreType.DMA,
        pltpu.SemaphoreType.DMA,
    ],
)


ROW_BLK = 1024


def _mlp_body(agg_ref, x_ref, wio_ref, wx_ref, b_ref, o_ref):
    base = jnp.dot(x_ref[...], wx_ref[...],
                   preferred_element_type=jnp.float32) + b_ref[...]
    ac = agg_ref[...].reshape(2 * D_EDGE, 8, 128)
    for j in range(8):
        aj = ac[:, j, :]
        part = lax.dot_general(aj, wio_ref[...], (((0,), (0,)), ((), ())),
                               preferred_element_type=jnp.float32)
        o_ref[pl.ds(j * 128, 128), :] = base[j * 128:(j + 1) * 128, :] + part


@functools.partial(jax.jit, static_argnames=())
def kernel(x, edge_attr, W, b, edge_index):
    e4 = edge_attr.T.reshape(NC, 8, N_EB, 128).transpose(0, 2, 1, 3)
    idx3 = edge_index.astype(jnp.int32).reshape(2, N_EB, 128)
    agg = _sc_agg(e4, idx3)

    wio = W[:2 * D_EDGE]
    wx = W[2 * D_EDGE:]
    b2 = b.reshape(1, D_FEAT)

    grid = (N_PAD // ROW_BLK,)
    out = pl.pallas_call(
        _mlp_body,
        grid=grid,
        in_specs=[
            pl.BlockSpec((NC, NS, 8, 128), lambda i: (0, 0, i, 0)),
            pl.BlockSpec((ROW_BLK, D_FEAT), lambda i: (i, 0)),
            pl.BlockSpec((2 * D_EDGE, D_FEAT), lambda i: (0, 0)),
            pl.BlockSpec((D_FEAT, D_FEAT), lambda i: (0, 0)),
            pl.BlockSpec((1, D_FEAT), lambda i: (0, 0)),
        ],
        out_specs=pl.BlockSpec((ROW_BLK, D_FEAT), lambda i: (i, 0)),
        out_shape=jax.ShapeDtypeStruct((N_NODES, D_FEAT), jnp.float32),
    )(agg, x, wio, wx, b2)
    return out


# speedup vs baseline: 18.6707x; 1.0107x over previous
import functools

import jax
import jax.numpy as jnp
from jax import lax
from jax.experimental import pallas as pl
from jax.experimental.pallas import tpu as pltpu
from jax.experimental.pallas import tpu_sc as plsc

N_NODES = 10000
N_EDGES = 320000
D_FEAT = 128
D_EDGE = 16

NC = 2
NS = 16

N_EB = N_EDGES // 128
CH = 125
N_CH = N_EB // CH
N_PAD = 10240
NRB = N_PAD // 128


def _sc_body(e4_hbm, idx_hbm, out_hbm,
             idx_sh, val_v, idx_v, acc, stg, vs0, vs1, is0, is1):
    c = lax.axis_index("c")
    s = lax.axis_index("s")
    fh = s // 8
    fm = lax.rem(s, 8)

    @pl.when(s == 0)
    def _():
        pltpu.sync_copy(idx_hbm, idx_sh)

    def _zero(i, carry):
        acc[pl.ds(i * 16, 16)] = jnp.zeros((16,), jnp.float32)
        return carry
    lax.fori_loop(0, N_PAD // 16, _zero, 0)
    plsc.subcore_barrier()

    vsems = (vs0, vs1)
    isems = (is0, is1)

    def _fetch(ch, b):
        pltpu.async_copy(
            e4_hbm.at[fh, pl.ds(ch * CH, CH), pl.ds(fm, 1), :],
            val_v.at[b], vsems[b])
        pltpu.async_copy(idx_sh.at[pl.ds(ch * CH, CH), pl.ds(1 - c, 1), :],
                         idx_v.at[b], isems[b])

    def _process(b):
        pltpu.make_async_copy(
            e4_hbm.at[fh, pl.ds(0, CH), pl.ds(fm, 1), :],
            val_v.at[b], vsems[b]).wait()
        pltpu.make_async_copy(idx_sh.at[pl.ds(0, CH), pl.ds(0, 1), :],
                              idx_v.at[b], isems[b]).wait()

        def _row(r, carry):
            ivs = [idx_v[b, r, 0, pl.ds(u * 16, 16)] for u in range(8)]
            vvs = [val_v[b, r, 0, pl.ds(u * 16, 16)] for u in range(8)]
            for u in range(8):
                plsc.addupdate_scatter(acc, [ivs[u]], vvs[u])
            return carry
        lax.fori_loop(0, CH, _row, 0)

    _fetch(0, 0)
    _fetch(1, 1)

    def _pair(t, carry):
        _process(0)

        @pl.when(2 * t + 2 < N_CH)
        def _():
            _fetch(2 * t + 2, 0)
        _process(1)

        @pl.when(2 * t + 3 < N_CH)
        def _():
            _fetch(2 * t + 3, 1)
        return carry
    lax.fori_loop(0, N_CH // 2, _pair, 0)

    def _rp(r, carry):
        for u in range(8):
            stg[r, pl.ds(u * 16, 16)] = acc[pl.ds(r * 128 + u * 16, 16)]
        return carry
    lax.fori_loop(0, NRB, _rp, 0)
    pltpu.sync_copy(stg, out_hbm.at[c, s])


_sc_agg = pl.kernel(
    _sc_body,
    out_type=jax.ShapeDtypeStruct((NC, NS, NRB, 128), jnp.float32),
    mesh=plsc.VectorSubcoreMesh(core_axis_name="c", subcore_axis_name="s"),
    compiler_params=pltpu.CompilerParams(use_tc_tiling_on_sc=False,
                                         needs_layout_passes=False),
    scratch_types=[
        pltpu.VMEM_SHARED((N_EB, 2, 128), jnp.int32),
        pltpu.VMEM((2, CH, 1, 128), jnp.float32),
        pltpu.VMEM((2, CH, 1, 128), jnp.int32),
        pltpu.VMEM((N_PAD,), jnp.float32),
        pltpu.VMEM((NRB, 128), jnp.float32),
        pltpu.SemaphoreType.DMA,
        pltpu.SemaphoreType.DMA,
        pltpu.SemaphoreType.DMA,
        pltpu.SemaphoreType.DMA,
    ],
)


ROW_BLK = 1024


def _mlp_body(agg_ref, x_ref, wio_ref, wx_ref, b_ref, o_ref):
    base = jnp.dot(x_ref[...], wx_ref[...],
                   preferred_element_type=jnp.float32) + b_ref[...]
    ac = agg_ref[...].reshape(2 * D_EDGE, 8, 128)
    for j in range(8):
        aj = ac[:, j, :]
        part = lax.dot_general(aj, wio_ref[...], (((0,), (0,)), ((), ())),
                               preferred_element_type=jnp.float32)
        o_ref[pl.ds(j * 128, 128), :] = base[j * 128:(j + 1) * 128, :] + part


@functools.partial(jax.jit, static_argnames=())
def kernel(x, edge_attr, W, b, edge_index):
    e4 = edge_attr.T.reshape(NC, 8, N_EB, 128).transpose(0, 2, 1, 3)
    idx3 = edge_index.astype(jnp.int32).reshape(2, N_EB, 128).transpose(1, 0, 2)
    agg = _sc_agg(e4, idx3)

    wio = W[:2 * D_EDGE]
    wx = W[2 * D_EDGE:]
    b2 = b.reshape(1, D_FEAT)

    grid = (N_PAD // ROW_BLK,)
    out = pl.pallas_call(
        _mlp_body,
        grid=grid,
        in_specs=[
            pl.BlockSpec((NC, NS, 8, 128), lambda i: (0, 0, i, 0)),
            pl.BlockSpec((ROW_BLK, D_FEAT), lambda i: (i, 0)),
            pl.BlockSpec((2 * D_EDGE, D_FEAT), lambda i: (0, 0)),
            pl.BlockSpec((D_FEAT, D_FEAT), lambda i: (0, 0)),
            pl.BlockSpec((1, D_FEAT), lambda i: (0, 0)),
        ],
        out_specs=pl.BlockSpec((ROW_BLK, D_FEAT), lambda i: (i, 0)),
        out_shape=jax.ShapeDtypeStruct((N_NODES, D_FEAT), jnp.float32),
    )(agg, x, wio, wx, b2)
    return out
